# baseline (speedup 1.0000x reference)
import functools

import jax
import jax.numpy as jnp
from jax.experimental import pallas as pl
from jax.experimental.pallas import tpu as pltpu

_TNC = 32768
_SUB = 8192
_NBUF = 5


def _ring_body(xt_hbm, w1_r, b1_r, w2_r, b2_r, o_hbm,
               xbuf, obuf, in_sems, out_sems, *, lanes_half, chunks):
    core = pl.program_id(0)
    base0 = core * lanes_half

    def in_copy(c, slot):
        return pltpu.make_async_copy(
            xt_hbm.at[:, pl.ds(base0 + c * _TNC, _TNC)],
            xbuf.at[slot], in_sems.at[slot])

    def out_copy(c, slot):
        return pltpu.make_async_copy(
            obuf.at[slot],
            o_hbm.at[:, pl.ds(base0 + c * _TNC, _TNC)], out_sems.at[slot])

    for s in range(min(_NBUF, chunks)):
        in_copy(s, s).start()

    def step(i, carry):
        slot = i % _NBUF

        if i >= _NBUF:
            out_copy(i - _NBUF, slot).wait()

        in_copy(i, slot).wait()
        for start in range(0, _TNC, _SUB):
            sl = pl.ds(start, _SUB)
            xc = xbuf[slot, :, sl].astype(jnp.bfloat16)
            h = jax.lax.dot_general(
                w1_r[...], xc,
                dimension_numbers=(((1,), (0,)), ((), ())),
                preferred_element_type=jnp.float32,
            )
            h = jnp.maximum(h + b1_r[...], 0.0)
            o = jax.lax.dot_general(
                w2_r[...], h.astype(jnp.bfloat16),
                dimension_numbers=(((1,), (0,)), ((), ())),
                preferred_element_type=jnp.float32,
            )
            obuf[slot, :, sl] = o + b2_r[...]
        out_copy(i, slot).start()

        if i + _NBUF < chunks:
            in_copy(i + _NBUF, slot).start()

        return carry

    for i in range(chunks):
        step(i, 0)
    for s in range(min(_NBUF, chunks)):
        out_copy(0, s).wait()


def _block_body(xt_ref, w1_ref, b1_ref, w2_ref, b2_ref, o_ref):
    tn = xt_ref.shape[1]
    for start in range(0, tn, _SUB):
        sl = pl.ds(start, min(_SUB, tn - start))
        xc = xt_ref[:, sl]
        h = jax.lax.dot_general(
            w1_ref[...], xc,
            dimension_numbers=(((1,), (0,)), ((), ())),
            preferred_element_type=jnp.float32,
        )
        h = jnp.maximum(h + b1_ref[...], 0.0)
        o = jax.lax.dot_general(
            w2_ref[...], h,
            dimension_numbers=(((1,), (0,)), ((), ())),
            preferred_element_type=jnp.float32,
        )
        o_ref[:, sl] = o + b2_ref[...]


def _block_kernel(xt, w1, b1c, w2, b2c, B, C):
    L, H = xt.shape[0], w1.shape[0]
    tn = 32768 if B % 32768 == 0 else B
    return pl.pallas_call(
        _block_body,
        out_shape=jax.ShapeDtypeStruct((C, B), xt.dtype),
        grid=(pl.cdiv(B, tn),),
        in_specs=[
            pl.BlockSpec((L, tn), lambda i: (0, i)),
            pl.BlockSpec((H, L), lambda i: (0, 0)),
            pl.BlockSpec((H, 1), lambda i: (0, 0)),
            pl.BlockSpec((C, H), lambda i: (0, 0)),
            pl.BlockSpec((C, 1), lambda i: (0, 0)),
        ],
        out_specs=pl.BlockSpec((C, tn), lambda i: (0, i)),
        compiler_params=pltpu.CompilerParams(
            dimension_semantics=("parallel",),
            vmem_limit_bytes=64 * 1024 * 1024,
        ),
    )(xt, w1, b1c, w2, b2c)


def kernel(x, w1, b1, w2, b2):
    B, L = x.shape
    H = w1.shape[0]
    C = w2.shape[0]

    xt = jnp.transpose(x)
    b1c = b1.reshape(H, 1)
    b2c = b2.reshape(C, 1)

    n_cores = 1
    if B % (n_cores * _TNC) != 0:
        return jnp.transpose(_block_kernel(xt, w1, b1c, w2, b2c, B, C))
    lanes_half = B // n_cores
    chunks = lanes_half // _TNC

    body = functools.partial(_ring_body, lanes_half=lanes_half,
                             chunks=chunks)
    w1b = w1.astype(jnp.bfloat16)
    w2b = w2.astype(jnp.bfloat16)

    ot = pl.pallas_call(
        body,
        out_shape=jax.ShapeDtypeStruct((C, B), x.dtype),
        grid=(n_cores,),
        in_specs=[
            pl.BlockSpec(memory_space=pl.ANY),
            pl.BlockSpec((H, L), lambda i: (0, 0)),
            pl.BlockSpec((H, 1), lambda i: (0, 0)),
            pl.BlockSpec((C, H), lambda i: (0, 0)),
            pl.BlockSpec((C, 1), lambda i: (0, 0)),
        ],
        out_specs=pl.BlockSpec(memory_space=pl.ANY),
        scratch_shapes=[
            pltpu.VMEM((_NBUF, L, _TNC), jnp.float32),
            pltpu.VMEM((_NBUF, C, _TNC), jnp.float32),
            pltpu.SemaphoreType.DMA((_NBUF,)),
            pltpu.SemaphoreType.DMA((_NBUF,)),
        ],
        compiler_params=pltpu.CompilerParams(
            dimension_semantics=("parallel",),
            vmem_limit_bytes=64 * 1024 * 1024,
        ),
    )(xt, w1b, b1c, w2b, b2c)
    return jnp.transpose(ot)


# speedup vs baseline: 1.0154x; 1.0154x over previous
import functools

import jax
import jax.numpy as jnp
from jax.experimental import pallas as pl
from jax.experimental.pallas import tpu as pltpu

_TNC = 65536
_SUB = 8192
_NBUF = 4


def _ring_body(xt_hbm, w1_r, b1_r, w2_r, b2_r, o_hbm,
               xbuf, obuf, in_sems, out_sems, *, lanes_half, chunks):
    core = pl.program_id(0)
    base0 = core * lanes_half

    def in_copy(c, slot):
        return pltpu.make_async_copy(
            xt_hbm.at[:, pl.ds(base0 + c * _TNC, _TNC)],
            xbuf.at[slot], in_sems.at[slot])

    def out_copy(c, slot):
        return pltpu.make_async_copy(
            obuf.at[slot],
            o_hbm.at[:, pl.ds(base0 + c * _TNC, _TNC)], out_sems.at[slot])

    for s in range(min(_NBUF, chunks)):
        in_copy(s, s).start()

    def step(i, carry):
        slot = i % _NBUF

        if i >= _NBUF:
            out_copy(i - _NBUF, slot).wait()

        in_copy(i, slot).wait()
        for start in range(0, _TNC, _SUB):
            sl = pl.ds(start, _SUB)
            xc = xbuf[slot, :, sl].astype(jnp.bfloat16)
            h = jax.lax.dot_general(
                w1_r[...], xc,
                dimension_numbers=(((1,), (0,)), ((), ())),
                preferred_element_type=jnp.float32,
            )
            h = jnp.maximum(h + b1_r[...], 0.0)
            o = jax.lax.dot_general(
                w2_r[...], h.astype(jnp.bfloat16),
                dimension_numbers=(((1,), (0,)), ((), ())),
                preferred_element_type=jnp.float32,
            )
            obuf[slot, :, sl] = o + b2_r[...]
        out_copy(i, slot).start()

        if i + _NBUF < chunks:
            in_copy(i + _NBUF, slot).start()

        return carry

    for i in range(chunks):
        step(i, 0)
    for s in range(min(_NBUF, chunks)):
        out_copy(0, s).wait()


def _block_body(xt_ref, w1_ref, b1_ref, w2_ref, b2_ref, o_ref):
    tn = xt_ref.shape[1]
    for start in range(0, tn, _SUB):
        sl = pl.ds(start, min(_SUB, tn - start))
        xc = xt_ref[:, sl]
        h = jax.lax.dot_general(
            w1_ref[...], xc,
            dimension_numbers=(((1,), (0,)), ((), ())),
            preferred_element_type=jnp.float32,
        )
        h = jnp.maximum(h + b1_ref[...], 0.0)
        o = jax.lax.dot_general(
            w2_ref[...], h,
            dimension_numbers=(((1,), (0,)), ((), ())),
            preferred_element_type=jnp.float32,
        )
        o_ref[:, sl] = o + b2_ref[...]


def _block_kernel(xt, w1, b1c, w2, b2c, B, C):
    L, H = xt.shape[0], w1.shape[0]
    tn = 32768 if B % 32768 == 0 else B
    return pl.pallas_call(
        _block_body,
        out_shape=jax.ShapeDtypeStruct((C, B), xt.dtype),
        grid=(pl.cdiv(B, tn),),
        in_specs=[
            pl.BlockSpec((L, tn), lambda i: (0, i)),
            pl.BlockSpec((H, L), lambda i: (0, 0)),
            pl.BlockSpec((H, 1), lambda i: (0, 0)),
            pl.BlockSpec((C, H), lambda i: (0, 0)),
            pl.BlockSpec((C, 1), lambda i: (0, 0)),
        ],
        out_specs=pl.BlockSpec((C, tn), lambda i: (0, i)),
        compiler_params=pltpu.CompilerParams(
            dimension_semantics=("parallel",),
            vmem_limit_bytes=64 * 1024 * 1024,
        ),
    )(xt, w1, b1c, w2, b2c)


def kernel(x, w1, b1, w2, b2):
    B, L = x.shape
    H = w1.shape[0]
    C = w2.shape[0]

    xt = jnp.transpose(x)
    b1c = b1.reshape(H, 1)
    b2c = b2.reshape(C, 1)

    n_cores = 1
    if B % (n_cores * _TNC) != 0:
        return jnp.transpose(_block_kernel(xt, w1, b1c, w2, b2c, B, C))
    lanes_half = B // n_cores
    chunks = lanes_half // _TNC

    body = functools.partial(_ring_body, lanes_half=lanes_half,
                             chunks=chunks)
    w1b = w1.astype(jnp.bfloat16)
    w2b = w2.astype(jnp.bfloat16)

    ot = pl.pallas_call(
        body,
        out_shape=jax.ShapeDtypeStruct((C, B), x.dtype),
        grid=(n_cores,),
        in_specs=[
            pl.BlockSpec(memory_space=pl.ANY),
            pl.BlockSpec((H, L), lambda i: (0, 0)),
            pl.BlockSpec((H, 1), lambda i: (0, 0)),
            pl.BlockSpec((C, H), lambda i: (0, 0)),
            pl.BlockSpec((C, 1), lambda i: (0, 0)),
        ],
        out_specs=pl.BlockSpec(memory_space=pl.ANY),
        scratch_shapes=[
            pltpu.VMEM((_NBUF, L, _TNC), jnp.float32),
            pltpu.VMEM((_NBUF, C, _TNC), jnp.float32),
            pltpu.SemaphoreType.DMA((_NBUF,)),
            pltpu.SemaphoreType.DMA((_NBUF,)),
        ],
        compiler_params=pltpu.CompilerParams(
            dimension_semantics=("parallel",),
            vmem_limit_bytes=64 * 1024 * 1024,
        ),
    )(xt, w1b, b1c, w2b, b2c)
    return jnp.transpose(ot)
